# baseline (device time: 19355 ns/iter reference)
import jax
import jax.numpy as jnp
from jax import lax
from jax.experimental import pallas as pl
from jax.experimental.pallas import tpu as pltpu

N_DEV = 4


def kernel(x, W1, W2):
    m, k = x.shape
    h_per = W1.shape[1]
    n = W2.shape[1]
    gw = n // 2
    hh = m // 2

    def body(x_ref, w1_ref, w2_ref, out_ref,
             p_ref, half_ref, full_ref, recv_ref, send_sems, recv_sems):
        my_pos = lax.axis_index("i")
        pa = my_pos ^ 1
        pb = (N_DEV - 1) - my_pos

        barrier_sem = pltpu.get_barrier_semaphore()
        for nbr in (pa, pb):
            pl.semaphore_signal(
                barrier_sem, inc=1,
                device_id=(nbr,), device_id_type=pl.DeviceIdType.MESH,
            )
        pl.semaphore_wait(barrier_sem, 2)

        partners = [(pa, pb, pa), (pb, pa, pb)]
        keep_top = [
            (my_pos == 0) | (my_pos == 3),
            my_pos <= 1,
        ]
        k_off = [jnp.where(kt, 0, hh) for kt in keep_top]
        s_off = [hh - ko for ko in k_off]

        xb = x_ref[...].astype(jnp.bfloat16)
        w1b = w1_ref[...].astype(jnp.bfloat16)
        hb = jnp.maximum(
            jnp.dot(xb, w1b, preferred_element_type=jnp.float32), 0.0
        ).astype(jnp.bfloat16)
        w2b = w2_ref[...].astype(jnp.bfloat16)

        def slot(stage, g):
            return stage * 2 + g

        def copy(stage, g, src):
            return pltpu.make_async_remote_copy(
                src_ref=src,
                dst_ref=recv_ref.at[slot(stage, g)],
                send_sem=send_sems.at[slot(stage, g)],
                recv_sem=recv_sems.at[slot(stage, g)],
                device_id=(partners[g][stage],),
                device_id_type=pl.DeviceIdType.MESH,
            )

        rdma0 = []
        for g in range(2):
            pc = jnp.dot(
                hb, w2b[:, g * gw:(g + 1) * gw],
                preferred_element_type=jnp.float32,
            )
            p_ref[g] = pc.astype(jnp.bfloat16)
            r = copy(0, g, p_ref.at[g, pl.ds(s_off[g], hh), :])
            r.start()
            rdma0.append(r)

        rdma1 = []
        for g in range(2):
            rdma0[g].wait_recv()
            half_ref[g] = (
                p_ref[g, pl.ds(k_off[g], hh), :] + recv_ref[slot(0, g)]
            )
            r = copy(1, g, half_ref.at[g])
            r.start()
            rdma1.append(r)

        rdma2 = []
        for g in range(2):
            rdma1[g].wait_recv()
            full = half_ref[g] + recv_ref[slot(1, g)]
            full_ref[g] = full
            r = copy(2, g, full_ref.at[g])
            r.start()
            rdma2.append(r)
            out_ref[pl.ds(k_off[g], hh), g * gw:(g + 1) * gw] = (
                full.astype(jnp.float32)
            )

        for g in range(2):
            rdma2[g].wait_recv()
            out_ref[pl.ds(s_off[g], hh), g * gw:(g + 1) * gw] = (
                recv_ref[slot(2, g)].astype(jnp.float32)
            )

        for r in rdma0 + rdma1 + rdma2:
            r.wait_send()

    return pl.pallas_call(
        body,
        out_shape=jax.ShapeDtypeStruct((m, n), jnp.float32),
        in_specs=[
            pl.BlockSpec(memory_space=pltpu.VMEM),
            pl.BlockSpec(memory_space=pltpu.VMEM),
            pl.BlockSpec(memory_space=pltpu.VMEM),
        ],
        out_specs=pl.BlockSpec(memory_space=pltpu.VMEM),
        scratch_shapes=[
            pltpu.VMEM((2, m, gw), jnp.bfloat16),
            pltpu.VMEM((2, hh, gw), jnp.bfloat16),
            pltpu.VMEM((2, hh, gw), jnp.bfloat16),
            pltpu.VMEM((6, hh, gw), jnp.bfloat16),
            pltpu.SemaphoreType.DMA((6,)),
            pltpu.SemaphoreType.DMA((6,)),
        ],
        compiler_params=pltpu.CompilerParams(collective_id=0),
    )(x, W1, W2)
